# baseline (speedup 1.0000x reference)
import functools

import jax
import jax.numpy as jnp
from jax import lax
from jax.experimental import pallas as pl
from jax.experimental.pallas import tpu as pltpu
from jax.experimental.pallas import tpu_sc as plsc

RISKS = ("risk_a", "risk_b", "risk_c")
IN_MAP = {"risk_a": (0, 1), "risk_b": (1, 2), "risk_c": (0, 2, 3)}
D = 32
H = 2
NS = 16
LANES = 16

BN = 1000


def _gelu(x):
    return 0.5 * x * (1.0 + lax.erf(x * (2.0 ** -0.5)))


def _lrelu(x):
    return jnp.where(x >= 0, x, 0.2 * x)


def _prepass_body(m1, m2, m3, m4,
                  w1a, b1a, w2a, b2a, w1b, b1b, w2b, b2b, w1c, b1c, w2c, b2c,
                  wl, wr,
                  lat_a, lat_b, lat_c, xl6, xr6):
    mods = (m1[...], m2[...], m3[...], m4[...])
    encs = ((w1a, b1a, w2a, b2a), (w1b, b1b, w2b, b2b), (w1c, b1c, w2c, b2c))
    lat_refs = (lat_a, lat_b, lat_c)
    for r in range(3):
        w1, b1, w2, b2 = encs[r]
        h = b1[...]
        for k, col in enumerate(IN_MAP[RISKS[r]]):
            h = h + mods[col] * w1[k:k + 1, :]
        h = _gelu(h)
        lat = jnp.dot(h, w2[...], precision=lax.Precision.HIGHEST) + b2[...]
        lat_refs[r][...] = lat
        xl = jnp.dot(lat, wl[...], precision=lax.Precision.HIGHEST)
        xr = jnp.dot(lat, wr[...], precision=lax.Precision.HIGHEST)
        xl6[2 * r, :, :] = xl[:, :D]
        xl6[2 * r + 1, :, :] = xl[:, D:]
        xr6[2 * r, :, :] = xr[:, :D]
        xr6[2 * r + 1, :, :] = xr[:, D:]


def _run_prepass(mods, params, n):
    grid = n // BN
    full = lambda shape: pl.BlockSpec(shape, lambda i: tuple(0 for _ in shape))
    in_specs = [pl.BlockSpec((BN, 1), lambda i: (i, 0))] * 4
    wargs = []
    for risk in RISKS:
        p = params[risk]
        wargs += [p["enc_W1"], p["enc_b1"].reshape(1, 64),
                  p["enc_W2"], p["enc_b2"].reshape(1, D)]
    g = params["gat"]
    wargs += [g["Wl"], g["Wr"]]
    in_specs += [full(w.shape) for w in wargs]
    out_shape = [jax.ShapeDtypeStruct((n, D), jnp.float32) for _ in range(3)]
    out_shape += [jax.ShapeDtypeStruct((6, n, D), jnp.float32)] * 2
    out_specs = [pl.BlockSpec((BN, D), lambda i: (i, 0))] * 3
    out_specs += [pl.BlockSpec((6, BN, D), lambda i: (0, i, 0))] * 2
    return pl.pallas_call(
        _prepass_body,
        grid=(grid,),
        in_specs=in_specs,
        out_specs=out_specs,
        out_shape=out_shape,
    )(*mods, *wargs)


def _make_sc_kernel(n, ep, nrows, nchunks):
    rows_pt = nchunks * 2
    stripe = 3120
    mesh = plsc.VectorSubcoreMesh(core_axis_name="c", subcore_axis_name="s")
    f32 = jnp.float32

    @functools.partial(
        pl.kernel,
        mesh=mesh,
        compiler_params=pltpu.CompilerParams(needs_layout_passes=False,
                                             use_tc_tiling_on_sc=False),
        out_type=[jax.ShapeDtypeStruct((6 * n, D), f32),
                  jax.ShapeDtypeStruct((6 * n,), f32)],
        scratch_types=[
            pltpu.VMEM((2, 128), jnp.int32),
            pltpu.VMEM((2, 128), jnp.int32),
            pltpu.VMEM((2, 128, D), f32),
            pltpu.VMEM((2, 128, D), f32),
            pltpu.VMEM((2, 128), f32),
            pltpu.VMEM((D,), f32),
            pltpu.VMEM((40, D), f32),
            pltpu.VMEM((48,), f32),
            pltpu.VMEM_SHARED((nrows, D), f32),
            pltpu.VMEM_SHARED((nrows,), f32),
            pltpu.SemaphoreType.DMA,
            pltpu.SemaphoreType.DMA,
        ],
    )
    def sc_edges(xl6, xr6, srcg, dstg, attsc,
                 num_out, den_out,
                 idx_s, idx_dg, xlr, xrr, exb, attv,
                 zbuf, zden, num_sh, den_sh, sem1, sem2):
        c = lax.axis_index("c")
        s = lax.axis_index("s")
        z16 = jnp.zeros((LANES,), f32)

        for i in range(40):
            zbuf[i, pl.ds(0, LANES)] = z16
            zbuf[i, pl.ds(LANES, LANES)] = z16
        for i in range(3):
            zden[pl.ds(i * LANES, LANES)] = z16

        pltpu.sync_copy(attsc.at[c], attv)

        lane = lax.iota(jnp.int32, LANES)
        zero_base = stripe * s

        def zero_body(k, _):
            pltpu.sync_copy(zbuf, num_sh.at[pl.ds(zero_base + k * 40, 40)])
            pltpu.sync_copy(zden.at[pl.ds(0, 40)],
                            den_sh.at[pl.ds(zero_base + k * 40, 40)])
            return 0

        for r in range(3):
            lax.fori_loop(0, 80, zero_body, 0)
            plsc.subcore_barrier()

            t_vec = jnp.full((LANES,), (2 * r) * n, jnp.int32) + c * n

            def chunk_body(j, _):
                row0 = s * rows_pt + j * 2
                pltpu.sync_copy(srcg.at[r, c, pl.ds(row0, 2)], idx_s)
                pltpu.sync_copy(dstg.at[r, c, pl.ds(row0, 2)], idx_dg)
                for b in range(2):
                    cp1 = pltpu.async_copy(xl6.at[idx_s.at[b]], xlr.at[b], sem1)
                    cp2 = pltpu.async_copy(xr6.at[idx_dg.at[b]], xrr.at[b], sem2)
                    cp1.wait()
                    cp2.wait()

                    def unoff_body(g, _):
                        sl = pl.ds(g * LANES, LANES)
                        idx_dg[b, sl] = idx_dg[b, sl] - t_vec
                        return 0

                    lax.fori_loop(0, 8, unoff_body, 0)

                    att0 = attv[pl.ds(0, LANES)]
                    att1 = attv[pl.ds(LANES, LANES)]
                    bb = jnp.full((LANES,), b, jnp.int32)
                    last = jnp.full((LANES,), LANES - 1, jnp.int32)
                    m0 = lane == 0

                    def edge_body(e, _):
                        xl0 = xlr[b, e, pl.ds(0, LANES)]
                        xl1 = xlr[b, e, pl.ds(LANES, LANES)]
                        xr0 = xrr[b, e, pl.ds(0, LANES)]
                        xr1 = xrr[b, e, pl.ds(LANES, LANES)]
                        z0 = xl0 + xr0
                        z1 = xl1 + xr1
                        t = (jnp.maximum(z0, 0.2 * z0) * att0
                             + jnp.maximum(z1, 0.2 * z1) * att1)
                        tot = jnp.take_along_axis(plsc.cumsum(t), last,
                                                  axis=0)
                        ex = jnp.exp(tot)
                        xrr[b, e, pl.ds(0, LANES)] = xl0 * ex
                        xrr[b, e, pl.ds(LANES, LANES)] = xl1 * ex
                        plsc.store_scatter(
                            exb, [bb, jnp.full((LANES,), e, jnp.int32)],
                            ex, mask=m0)
                        return 0

                    lax.fori_loop(0, 128, edge_body, 0, unroll=4)
                    pltpu.sync_copy(xrr.at[b], num_sh.at[idx_dg.at[b]],
                                    add=True)
                    pltpu.sync_copy(exb.at[b], den_sh.at[idx_dg.at[b]],
                                    add=True)
                return 0

            lax.fori_loop(0, nchunks, chunk_body, 0)
            plsc.subcore_barrier()

            t_off = (2 * r + c) * n
            b0 = stripe * s
            pltpu.sync_copy(num_sh.at[pl.ds(b0, stripe)],
                            num_out.at[pl.ds(t_off + b0, stripe)])
            pltpu.sync_copy(den_sh.at[pl.ds(b0, stripe)],
                            den_out.at[pl.ds(t_off + b0, stripe)])

            @pl.when(s == NS - 1)
            def _():
                pltpu.sync_copy(num_sh.at[pl.ds(15 * stripe + stripe, 80)],
                                num_out.at[pl.ds(t_off + 16 * stripe, 80)])
                pltpu.sync_copy(den_sh.at[pl.ds(15 * stripe + stripe, 80)],
                                den_out.at[pl.ds(t_off + 16 * stripe, 80)])

    return sc_edges


def _finalize_body(num, den, xl6, xr6, att64, bias,
                   hw1a, hb1a, hw2a, hb2a, hw1b, hb1b, hw2b, hb2b,
                   hw1c, hb1c, hw2c, hb2c, out):
    heads = ((hw1a, hb1a, hw2a, hb2a), (hw1b, hb1b, hw2b, hb2b),
             (hw1c, hb1c, hw2c, hb2c))
    preds = []
    for r in range(3):
        gs = []
        for h in range(H):
            t = 2 * r + h
            xl = xl6[t, :, :]
            z = xl + xr6[t, :, :]
            logit = jnp.sum(_lrelu(z) * att64[:, h * D:(h + 1) * D],
                            axis=1, keepdims=True)
            ex = jnp.exp(logit)
            dent = den[t, :, :] + ex
            numt = num[t, :, :] + xl * ex
            gs.append(numt / (dent + 1e-16))
        gout = 0.5 * (gs[0] + gs[1]) + bias[...]
        gact = _gelu(gout)
        hw1, hb1, hw2, hb2 = heads[r]
        h1 = _gelu(jnp.dot(gact, hw1[...],
                           precision=lax.Precision.HIGHEST) + hb1[...])
        z2 = jnp.dot(h1, hw2[...], precision=lax.Precision.HIGHEST) + hb2[...]
        preds.append(jax.nn.sigmoid(z2))
    out[...] = jnp.concatenate(preds, axis=1)


def _run_finalize(num6, den6, xl6, xr6, params, n):
    grid = n // BN
    full = lambda shape: pl.BlockSpec(shape, lambda i: tuple(0 for _ in shape))
    g = params["gat"]
    att64 = g["att"].reshape(1, H * D)
    bias = g["bias"].reshape(1, D)
    wargs = [att64, bias]
    for risk in RISKS:
        p = params[risk]
        wargs += [p["head_W1"], p["head_b1"].reshape(1, D),
                  p["head_W2"], p["head_b2"].reshape(1, 1)]
    num = num6.reshape(6, n, D)
    den = den6.reshape(6, n, 1)
    in_specs = [pl.BlockSpec((6, BN, D), lambda i: (0, i, 0)),
                pl.BlockSpec((6, BN, 1), lambda i: (0, i, 0)),
                pl.BlockSpec((6, BN, D), lambda i: (0, i, 0)),
                pl.BlockSpec((6, BN, D), lambda i: (0, i, 0))]
    in_specs += [full(w.shape) for w in wargs]
    return pl.pallas_call(
        _finalize_body,
        grid=(grid,),
        in_specs=in_specs,
        out_specs=pl.BlockSpec((BN, 3), lambda i: (i, 0)),
        out_shape=jax.ShapeDtypeStruct((n, 3), jnp.float32),
    )(num, den, xl6.reshape(6, n, D), xr6.reshape(6, n, D), *wargs)


def kernel(mod1, mod2, mod3, mod4, edge_index, params):
    n = mod1.shape[0]
    e = edge_index.shape[1]

    lat_a, lat_b, lat_c, xl6, xr6 = _run_prepass(
        (mod1, mod2, mod3, mod4), params, n)

    nchunks = -(-e // (NS * 256))
    ep = NS * 256 * nchunks
    pad = ep - e
    src = jnp.concatenate([edge_index[0],
                           jnp.zeros((pad,), edge_index.dtype)])
    dst_raw = edge_index[1]
    dst = jnp.concatenate([dst_raw,
                           jnp.full((pad,), n, edge_index.dtype)])
    offs = (jnp.arange(6, dtype=jnp.int32) * n).reshape(3, 2, 1)
    srcg = (src[None, None, :] + offs).reshape(3, 2, ep // 128, 128)
    dstg = (dst[None, None, :] + offs).reshape(3, 2, ep // 128, 128)

    xl6f = jnp.concatenate([xl6.reshape(6 * n, D),
                            jnp.zeros((8, D), jnp.float32)])
    xr6f = jnp.concatenate([xr6.reshape(6 * n, D),
                            jnp.zeros((8, D), jnp.float32)])
    attsc = params["gat"]["att"]

    nrows = n + 56
    sc = _make_sc_kernel(n, ep, nrows, nchunks)
    num6, den6 = sc(xl6f, xr6f, srcg, dstg, attsc)

    risk_vector = _run_finalize(num6, den6, xl6, xr6, params, n)
    return (risk_vector, lat_a, lat_b, lat_c)


# speedup vs baseline: 1.0030x; 1.0030x over previous
import functools

import jax
import jax.numpy as jnp
from jax import lax
from jax.experimental import pallas as pl
from jax.experimental.pallas import tpu as pltpu
from jax.experimental.pallas import tpu_sc as plsc

RISKS = ("risk_a", "risk_b", "risk_c")
IN_MAP = {"risk_a": (0, 1), "risk_b": (1, 2), "risk_c": (0, 2, 3)}
D = 32
H = 2
NS = 16
LANES = 16

BN = 1000


def _gelu(x):
    return 0.5 * x * (1.0 + lax.erf(x * (2.0 ** -0.5)))


def _lrelu(x):
    return jnp.where(x >= 0, x, 0.2 * x)


def _prepass_body(m1, m2, m3, m4,
                  w1a, b1a, w2a, b2a, w1b, b1b, w2b, b2b, w1c, b1c, w2c, b2c,
                  wl, wr,
                  lat_a, lat_b, lat_c, xl6, xr6):
    mods = (m1[...], m2[...], m3[...], m4[...])
    encs = ((w1a, b1a, w2a, b2a), (w1b, b1b, w2b, b2b), (w1c, b1c, w2c, b2c))
    lat_refs = (lat_a, lat_b, lat_c)
    for r in range(3):
        w1, b1, w2, b2 = encs[r]
        h = b1[...]
        for k, col in enumerate(IN_MAP[RISKS[r]]):
            h = h + mods[col] * w1[k:k + 1, :]
        h = _gelu(h)
        lat = jnp.dot(h, w2[...], precision=lax.Precision.HIGHEST) + b2[...]
        lat_refs[r][...] = lat
        xl = jnp.dot(lat, wl[...], precision=lax.Precision.HIGHEST)
        xr = jnp.dot(lat, wr[...], precision=lax.Precision.HIGHEST)
        xl6[2 * r, :, :] = xl[:, :D]
        xl6[2 * r + 1, :, :] = xl[:, D:]
        xr6[2 * r, :, :] = xr[:, :D]
        xr6[2 * r + 1, :, :] = xr[:, D:]


def _run_prepass(mods, params, n):
    grid = n // BN
    full = lambda shape: pl.BlockSpec(shape, lambda i: tuple(0 for _ in shape))
    in_specs = [pl.BlockSpec((BN, 1), lambda i: (i, 0))] * 4
    wargs = []
    for risk in RISKS:
        p = params[risk]
        wargs += [p["enc_W1"], p["enc_b1"].reshape(1, 64),
                  p["enc_W2"], p["enc_b2"].reshape(1, D)]
    g = params["gat"]
    wargs += [g["Wl"], g["Wr"]]
    in_specs += [full(w.shape) for w in wargs]
    out_shape = [jax.ShapeDtypeStruct((n, D), jnp.float32) for _ in range(3)]
    out_shape += [jax.ShapeDtypeStruct((6, n, D), jnp.float32)] * 2
    out_specs = [pl.BlockSpec((BN, D), lambda i: (i, 0))] * 3
    out_specs += [pl.BlockSpec((6, BN, D), lambda i: (0, i, 0))] * 2
    return pl.pallas_call(
        _prepass_body,
        grid=(grid,),
        in_specs=in_specs,
        out_specs=out_specs,
        out_shape=out_shape,
    )(*mods, *wargs)


def _make_sc_kernel(n, ep, nrows, nchunks):
    rows_pt = nchunks * 2
    stripe = 3120
    mesh = plsc.VectorSubcoreMesh(core_axis_name="c", subcore_axis_name="s")
    f32 = jnp.float32

    @functools.partial(
        pl.kernel,
        mesh=mesh,
        compiler_params=pltpu.CompilerParams(needs_layout_passes=False,
                                             use_tc_tiling_on_sc=False),
        out_type=[jax.ShapeDtypeStruct((6 * n, D), f32),
                  jax.ShapeDtypeStruct((6 * n,), f32)],
        scratch_types=[
            pltpu.VMEM((2, 128), jnp.int32),
            pltpu.VMEM((2, 128), jnp.int32),
            pltpu.VMEM((2, 128, D), f32),
            pltpu.VMEM((2, 128, D), f32),
            pltpu.VMEM((2, 128), f32),
            pltpu.VMEM((D,), f32),
            pltpu.VMEM((40, D), f32),
            pltpu.VMEM((48,), f32),
            pltpu.VMEM_SHARED((nrows, D), f32),
            pltpu.VMEM_SHARED((nrows,), f32),
            pltpu.SemaphoreType.DMA,
            pltpu.SemaphoreType.DMA,
        ],
    )
    def sc_edges(xl6, xr6, srcg, dstg, attsc,
                 num_out, den_out,
                 idx_s, idx_dg, xlr, xrr, exb, attv,
                 zbuf, zden, num_sh, den_sh, sem1, sem2):
        c = lax.axis_index("c")
        s = lax.axis_index("s")
        z16 = jnp.zeros((LANES,), f32)

        for i in range(40):
            zbuf[i, pl.ds(0, LANES)] = z16
            zbuf[i, pl.ds(LANES, LANES)] = z16
        for i in range(3):
            zden[pl.ds(i * LANES, LANES)] = z16

        pltpu.sync_copy(attsc.at[c], attv)

        lane = lax.iota(jnp.int32, LANES)
        zero_base = stripe * s

        def zero_body(k, _):
            pltpu.sync_copy(zbuf, num_sh.at[pl.ds(zero_base + k * 40, 40)])
            pltpu.sync_copy(zden.at[pl.ds(0, 40)],
                            den_sh.at[pl.ds(zero_base + k * 40, 40)])
            return 0

        for r in range(3):
            lax.fori_loop(0, 80, zero_body, 0)
            plsc.subcore_barrier()

            t_vec = jnp.full((LANES,), (2 * r) * n, jnp.int32) + c * n

            def chunk_body(j, _):
                row0 = s * rows_pt + j * 2
                pltpu.sync_copy(srcg.at[r, c, pl.ds(row0, 2)], idx_s)
                pltpu.sync_copy(dstg.at[r, c, pl.ds(row0, 2)], idx_dg)
                for b in range(2):
                    cp1 = pltpu.async_copy(xl6.at[idx_s.at[b]], xlr.at[b], sem1)
                    cp2 = pltpu.async_copy(xr6.at[idx_dg.at[b]], xrr.at[b], sem2)
                    cp1.wait()
                    cp2.wait()

                    def unoff_body(g, _):
                        sl = pl.ds(g * LANES, LANES)
                        idx_dg[b, sl] = idx_dg[b, sl] - t_vec
                        return 0

                    lax.fori_loop(0, 8, unoff_body, 0)

                    att0 = attv[pl.ds(0, LANES)]
                    att1 = attv[pl.ds(LANES, LANES)]
                    bb = jnp.full((LANES,), b, jnp.int32)
                    last = jnp.full((LANES,), LANES - 1, jnp.int32)
                    m0 = lane == 0

                    def edge_body(e, _):
                        xl0 = xlr[b, e, pl.ds(0, LANES)]
                        xl1 = xlr[b, e, pl.ds(LANES, LANES)]
                        xr0 = xrr[b, e, pl.ds(0, LANES)]
                        xr1 = xrr[b, e, pl.ds(LANES, LANES)]
                        z0 = xl0 + xr0
                        z1 = xl1 + xr1
                        t = (jnp.maximum(z0, 0.2 * z0) * att0
                             + jnp.maximum(z1, 0.2 * z1) * att1)
                        tot = jnp.take_along_axis(plsc.cumsum(t), last,
                                                  axis=0)
                        ex = jnp.exp(tot)
                        xrr[b, e, pl.ds(0, LANES)] = xl0 * ex
                        xrr[b, e, pl.ds(LANES, LANES)] = xl1 * ex
                        plsc.store_scatter(
                            exb, [bb, jnp.full((LANES,), e, jnp.int32)],
                            ex, mask=m0)
                        return 0

                    lax.fori_loop(0, 128, edge_body, 0, unroll=8)
                    pltpu.sync_copy(xrr.at[b], num_sh.at[idx_dg.at[b]],
                                    add=True)
                    pltpu.sync_copy(exb.at[b], den_sh.at[idx_dg.at[b]],
                                    add=True)
                return 0

            lax.fori_loop(0, nchunks, chunk_body, 0)
            plsc.subcore_barrier()

            t_off = (2 * r + c) * n
            b0 = stripe * s
            pltpu.sync_copy(num_sh.at[pl.ds(b0, stripe)],
                            num_out.at[pl.ds(t_off + b0, stripe)])
            pltpu.sync_copy(den_sh.at[pl.ds(b0, stripe)],
                            den_out.at[pl.ds(t_off + b0, stripe)])

            @pl.when(s == NS - 1)
            def _():
                pltpu.sync_copy(num_sh.at[pl.ds(15 * stripe + stripe, 80)],
                                num_out.at[pl.ds(t_off + 16 * stripe, 80)])
                pltpu.sync_copy(den_sh.at[pl.ds(15 * stripe + stripe, 80)],
                                den_out.at[pl.ds(t_off + 16 * stripe, 80)])

    return sc_edges


def _finalize_body(num, den, xl6, xr6, att64, bias,
                   hw1a, hb1a, hw2a, hb2a, hw1b, hb1b, hw2b, hb2b,
                   hw1c, hb1c, hw2c, hb2c, out):
    heads = ((hw1a, hb1a, hw2a, hb2a), (hw1b, hb1b, hw2b, hb2b),
             (hw1c, hb1c, hw2c, hb2c))
    preds = []
    for r in range(3):
        gs = []
        for h in range(H):
            t = 2 * r + h
            xl = xl6[t, :, :]
            z = xl + xr6[t, :, :]
            logit = jnp.sum(_lrelu(z) * att64[:, h * D:(h + 1) * D],
                            axis=1, keepdims=True)
            ex = jnp.exp(logit)
            dent = den[t, :, :] + ex
            numt = num[t, :, :] + xl * ex
            gs.append(numt / (dent + 1e-16))
        gout = 0.5 * (gs[0] + gs[1]) + bias[...]
        gact = _gelu(gout)
        hw1, hb1, hw2, hb2 = heads[r]
        h1 = _gelu(jnp.dot(gact, hw1[...],
                           precision=lax.Precision.HIGHEST) + hb1[...])
        z2 = jnp.dot(h1, hw2[...], precision=lax.Precision.HIGHEST) + hb2[...]
        preds.append(jax.nn.sigmoid(z2))
    out[...] = jnp.concatenate(preds, axis=1)


def _run_finalize(num6, den6, xl6, xr6, params, n):
    grid = n // BN
    full = lambda shape: pl.BlockSpec(shape, lambda i: tuple(0 for _ in shape))
    g = params["gat"]
    att64 = g["att"].reshape(1, H * D)
    bias = g["bias"].reshape(1, D)
    wargs = [att64, bias]
    for risk in RISKS:
        p = params[risk]
        wargs += [p["head_W1"], p["head_b1"].reshape(1, D),
                  p["head_W2"], p["head_b2"].reshape(1, 1)]
    num = num6.reshape(6, n, D)
    den = den6.reshape(6, n, 1)
    in_specs = [pl.BlockSpec((6, BN, D), lambda i: (0, i, 0)),
                pl.BlockSpec((6, BN, 1), lambda i: (0, i, 0)),
                pl.BlockSpec((6, BN, D), lambda i: (0, i, 0)),
                pl.BlockSpec((6, BN, D), lambda i: (0, i, 0))]
    in_specs += [full(w.shape) for w in wargs]
    return pl.pallas_call(
        _finalize_body,
        grid=(grid,),
        in_specs=in_specs,
        out_specs=pl.BlockSpec((BN, 3), lambda i: (i, 0)),
        out_shape=jax.ShapeDtypeStruct((n, 3), jnp.float32),
    )(num, den, xl6.reshape(6, n, D), xr6.reshape(6, n, D), *wargs)


def kernel(mod1, mod2, mod3, mod4, edge_index, params):
    n = mod1.shape[0]
    e = edge_index.shape[1]

    lat_a, lat_b, lat_c, xl6, xr6 = _run_prepass(
        (mod1, mod2, mod3, mod4), params, n)

    nchunks = -(-e // (NS * 256))
    ep = NS * 256 * nchunks
    pad = ep - e
    src = jnp.concatenate([edge_index[0],
                           jnp.zeros((pad,), edge_index.dtype)])
    dst_raw = edge_index[1]
    dst = jnp.concatenate([dst_raw,
                           jnp.full((pad,), n, edge_index.dtype)])
    offs = (jnp.arange(6, dtype=jnp.int32) * n).reshape(3, 2, 1)
    srcg = (src[None, None, :] + offs).reshape(3, 2, ep // 128, 128)
    dstg = (dst[None, None, :] + offs).reshape(3, 2, ep // 128, 128)

    xl6f = jnp.concatenate([xl6.reshape(6 * n, D),
                            jnp.zeros((8, D), jnp.float32)])
    xr6f = jnp.concatenate([xr6.reshape(6 * n, D),
                            jnp.zeros((8, D), jnp.float32)])
    attsc = params["gat"]["att"]

    nrows = n + 56
    sc = _make_sc_kernel(n, ep, nrows, nchunks)
    num6, den6 = sc(xl6f, xr6f, srcg, dstg, attsc)

    risk_vector = _run_finalize(num6, den6, xl6, xr6, params, n)
    return (risk_vector, lat_a, lat_b, lat_c)


# speedup vs baseline: 1.7891x; 1.7837x over previous
import functools

import jax
import jax.numpy as jnp
from jax import lax
from jax.experimental import pallas as pl
from jax.experimental.pallas import tpu as pltpu
from jax.experimental.pallas import tpu_sc as plsc

RISKS = ("risk_a", "risk_b", "risk_c")
IN_MAP = {"risk_a": (0, 1), "risk_b": (1, 2), "risk_c": (0, 2, 3)}
D = 32
H = 2
NS = 16
LANES = 16

BN = 1000


def _gelu(x):
    return 0.5 * x * (1.0 + lax.erf(x * (2.0 ** -0.5)))


def _lrelu(x):
    return jnp.where(x >= 0, x, 0.2 * x)


def _prepass_body(m1, m2, m3, m4,
                  w1a, b1a, w2a, b2a, w1b, b1b, w2b, b2b, w1c, b1c, w2c, b2c,
                  wl, wr,
                  lat_a, lat_b, lat_c, xl6, xr6):
    mods = (m1[...], m2[...], m3[...], m4[...])
    encs = ((w1a, b1a, w2a, b2a), (w1b, b1b, w2b, b2b), (w1c, b1c, w2c, b2c))
    lat_refs = (lat_a, lat_b, lat_c)
    for r in range(3):
        w1, b1, w2, b2 = encs[r]
        h = b1[...]
        for k, col in enumerate(IN_MAP[RISKS[r]]):
            h = h + mods[col] * w1[k:k + 1, :]
        h = _gelu(h)
        lat = jnp.dot(h, w2[...], precision=lax.Precision.HIGHEST) + b2[...]
        lat_refs[r][...] = lat
        xl = jnp.dot(lat, wl[...], precision=lax.Precision.HIGHEST)
        xr = jnp.dot(lat, wr[...], precision=lax.Precision.HIGHEST)
        xl6[2 * r, :, :] = xl[:, :D]
        xl6[2 * r + 1, :, :] = xl[:, D:]
        xr6[2 * r, :, :] = xr[:, :D]
        xr6[2 * r + 1, :, :] = xr[:, D:]


def _run_prepass(mods, params, n):
    grid = n // BN
    full = lambda shape: pl.BlockSpec(shape, lambda i: tuple(0 for _ in shape))
    in_specs = [pl.BlockSpec((BN, 1), lambda i: (i, 0))] * 4
    wargs = []
    for risk in RISKS:
        p = params[risk]
        wargs += [p["enc_W1"], p["enc_b1"].reshape(1, 64),
                  p["enc_W2"], p["enc_b2"].reshape(1, D)]
    g = params["gat"]
    wargs += [g["Wl"], g["Wr"]]
    in_specs += [full(w.shape) for w in wargs]
    out_shape = [jax.ShapeDtypeStruct((n, D), jnp.float32) for _ in range(3)]
    out_shape += [jax.ShapeDtypeStruct((6, n, D), jnp.float32)] * 2
    out_specs = [pl.BlockSpec((BN, D), lambda i: (i, 0))] * 3
    out_specs += [pl.BlockSpec((6, BN, D), lambda i: (0, i, 0))] * 2
    return pl.pallas_call(
        _prepass_body,
        grid=(grid,),
        in_specs=in_specs,
        out_specs=out_specs,
        out_shape=out_shape,
    )(*mods, *wargs)


def _make_sc_kernel(n, ep, nrows, nchunks):
    rows_pt = nchunks * 2
    stripe = 3120
    mesh = plsc.VectorSubcoreMesh(core_axis_name="c", subcore_axis_name="s")
    f32 = jnp.float32

    @functools.partial(
        pl.kernel,
        mesh=mesh,
        compiler_params=pltpu.CompilerParams(needs_layout_passes=False,
                                             use_tc_tiling_on_sc=False),
        out_type=[jax.ShapeDtypeStruct((6 * n, D), f32),
                  jax.ShapeDtypeStruct((6 * n,), f32)],
        scratch_types=[
            pltpu.VMEM((2, 128), jnp.int32),
            pltpu.VMEM((2, 128), jnp.int32),
            pltpu.VMEM((2, 128, D), f32),
            pltpu.VMEM((2, 128, D), f32),
            pltpu.VMEM((2, 128), f32),
            pltpu.VMEM((D,), f32),
            pltpu.VMEM((40, D), f32),
            pltpu.VMEM((48,), f32),
            pltpu.VMEM_SHARED((nrows, D), f32),
            pltpu.VMEM_SHARED((nrows,), f32),
            pltpu.SemaphoreType.DMA,
            pltpu.SemaphoreType.DMA,
        ],
    )
    def sc_edges(xl6, xr6, srcg, dstg, attsc,
                 num_out, den_out,
                 idx_s, idx_dg, xlr, xrr, exb, attv,
                 zbuf, zden, num_sh, den_sh, sem1, sem2):
        c = lax.axis_index("c")
        s = lax.axis_index("s")
        z16 = jnp.zeros((LANES,), f32)

        for i in range(40):
            zbuf[i, pl.ds(0, LANES)] = z16
            zbuf[i, pl.ds(LANES, LANES)] = z16
        for i in range(3):
            zden[pl.ds(i * LANES, LANES)] = z16

        pltpu.sync_copy(attsc.at[c], attv)

        lane = lax.iota(jnp.int32, LANES)
        zero_base = stripe * s

        def zero_body(k, _):
            pltpu.sync_copy(zbuf, num_sh.at[pl.ds(zero_base + k * 40, 40)])
            pltpu.sync_copy(zden.at[pl.ds(0, 40)],
                            den_sh.at[pl.ds(zero_base + k * 40, 40)])
            return 0

        for r in range(3):
            lax.fori_loop(0, 80, zero_body, 0)
            plsc.subcore_barrier()

            t_vec = jnp.full((LANES,), (2 * r) * n, jnp.int32) + c * n

            def chunk_body(j, _):
                row0 = s * rows_pt + j * 2
                pltpu.sync_copy(srcg.at[r, c, pl.ds(row0, 2)], idx_s)
                pltpu.sync_copy(dstg.at[r, c, pl.ds(row0, 2)], idx_dg)
                for b in range(2):
                    cp1 = pltpu.async_copy(xl6.at[idx_s.at[b]], xlr.at[b], sem1)
                    cp2 = pltpu.async_copy(xr6.at[idx_dg.at[b]], xrr.at[b], sem2)
                    cp1.wait()
                    cp2.wait()

                    def unoff_body(g, _):
                        sl = pl.ds(g * LANES, LANES)
                        idx_dg[b, sl] = idx_dg[b, sl] - t_vec
                        return 0

                    lax.fori_loop(0, 8, unoff_body, 0)

                    att0 = attv[pl.ds(0, LANES)]
                    att1 = attv[pl.ds(LANES, LANES)]
                    bb = jnp.full((LANES,), b, jnp.int32)
                    last = jnp.full((LANES,), LANES - 1, jnp.int32)
                    m0 = lane == 0

                    @plsc.parallel_loop(0, 128, unroll=8)
                    def edge_body(e):
                        xl0 = xlr[b, e, pl.ds(0, LANES)]
                        xl1 = xlr[b, e, pl.ds(LANES, LANES)]
                        xr0 = xrr[b, e, pl.ds(0, LANES)]
                        xr1 = xrr[b, e, pl.ds(LANES, LANES)]
                        z0 = xl0 + xr0
                        z1 = xl1 + xr1
                        t = (jnp.maximum(z0, 0.2 * z0) * att0
                             + jnp.maximum(z1, 0.2 * z1) * att1)
                        tot = jnp.take_along_axis(plsc.cumsum(t), last,
                                                  axis=0)
                        ex = jnp.exp(tot)
                        xrr[b, e, pl.ds(0, LANES)] = xl0 * ex
                        xrr[b, e, pl.ds(LANES, LANES)] = xl1 * ex
                        plsc.store_scatter(
                            exb, [bb, jnp.full((LANES,), e, jnp.int32)],
                            ex, mask=m0)

                    pltpu.sync_copy(xrr.at[b], num_sh.at[idx_dg.at[b]],
                                    add=True)
                    pltpu.sync_copy(exb.at[b], den_sh.at[idx_dg.at[b]],
                                    add=True)
                return 0

            lax.fori_loop(0, nchunks, chunk_body, 0)
            plsc.subcore_barrier()

            t_off = (2 * r + c) * n
            b0 = stripe * s
            pltpu.sync_copy(num_sh.at[pl.ds(b0, stripe)],
                            num_out.at[pl.ds(t_off + b0, stripe)])
            pltpu.sync_copy(den_sh.at[pl.ds(b0, stripe)],
                            den_out.at[pl.ds(t_off + b0, stripe)])

            @pl.when(s == NS - 1)
            def _():
                pltpu.sync_copy(num_sh.at[pl.ds(15 * stripe + stripe, 80)],
                                num_out.at[pl.ds(t_off + 16 * stripe, 80)])
                pltpu.sync_copy(den_sh.at[pl.ds(15 * stripe + stripe, 80)],
                                den_out.at[pl.ds(t_off + 16 * stripe, 80)])

    return sc_edges


def _finalize_body(num, den, xl6, xr6, att64, bias,
                   hw1a, hb1a, hw2a, hb2a, hw1b, hb1b, hw2b, hb2b,
                   hw1c, hb1c, hw2c, hb2c, out):
    heads = ((hw1a, hb1a, hw2a, hb2a), (hw1b, hb1b, hw2b, hb2b),
             (hw1c, hb1c, hw2c, hb2c))
    preds = []
    for r in range(3):
        gs = []
        for h in range(H):
            t = 2 * r + h
            xl = xl6[t, :, :]
            z = xl + xr6[t, :, :]
            logit = jnp.sum(_lrelu(z) * att64[:, h * D:(h + 1) * D],
                            axis=1, keepdims=True)
            ex = jnp.exp(logit)
            dent = den[t, :, :] + ex
            numt = num[t, :, :] + xl * ex
            gs.append(numt / (dent + 1e-16))
        gout = 0.5 * (gs[0] + gs[1]) + bias[...]
        gact = _gelu(gout)
        hw1, hb1, hw2, hb2 = heads[r]
        h1 = _gelu(jnp.dot(gact, hw1[...],
                           precision=lax.Precision.HIGHEST) + hb1[...])
        z2 = jnp.dot(h1, hw2[...], precision=lax.Precision.HIGHEST) + hb2[...]
        preds.append(jax.nn.sigmoid(z2))
    out[...] = jnp.concatenate(preds, axis=1)


def _run_finalize(num6, den6, xl6, xr6, params, n):
    grid = n // BN
    full = lambda shape: pl.BlockSpec(shape, lambda i: tuple(0 for _ in shape))
    g = params["gat"]
    att64 = g["att"].reshape(1, H * D)
    bias = g["bias"].reshape(1, D)
    wargs = [att64, bias]
    for risk in RISKS:
        p = params[risk]
        wargs += [p["head_W1"], p["head_b1"].reshape(1, D),
                  p["head_W2"], p["head_b2"].reshape(1, 1)]
    num = num6.reshape(6, n, D)
    den = den6.reshape(6, n, 1)
    in_specs = [pl.BlockSpec((6, BN, D), lambda i: (0, i, 0)),
                pl.BlockSpec((6, BN, 1), lambda i: (0, i, 0)),
                pl.BlockSpec((6, BN, D), lambda i: (0, i, 0)),
                pl.BlockSpec((6, BN, D), lambda i: (0, i, 0))]
    in_specs += [full(w.shape) for w in wargs]
    return pl.pallas_call(
        _finalize_body,
        grid=(grid,),
        in_specs=in_specs,
        out_specs=pl.BlockSpec((BN, 3), lambda i: (i, 0)),
        out_shape=jax.ShapeDtypeStruct((n, 3), jnp.float32),
    )(num, den, xl6.reshape(6, n, D), xr6.reshape(6, n, D), *wargs)


def kernel(mod1, mod2, mod3, mod4, edge_index, params):
    n = mod1.shape[0]
    e = edge_index.shape[1]

    lat_a, lat_b, lat_c, xl6, xr6 = _run_prepass(
        (mod1, mod2, mod3, mod4), params, n)

    nchunks = -(-e // (NS * 256))
    ep = NS * 256 * nchunks
    pad = ep - e
    src = jnp.concatenate([edge_index[0],
                           jnp.zeros((pad,), edge_index.dtype)])
    dst_raw = edge_index[1]
    dst = jnp.concatenate([dst_raw,
                           jnp.full((pad,), n, edge_index.dtype)])
    offs = (jnp.arange(6, dtype=jnp.int32) * n).reshape(3, 2, 1)
    srcg = (src[None, None, :] + offs).reshape(3, 2, ep // 128, 128)
    dstg = (dst[None, None, :] + offs).reshape(3, 2, ep // 128, 128)

    xl6f = jnp.concatenate([xl6.reshape(6 * n, D),
                            jnp.zeros((8, D), jnp.float32)])
    xr6f = jnp.concatenate([xr6.reshape(6 * n, D),
                            jnp.zeros((8, D), jnp.float32)])
    attsc = params["gat"]["att"]

    nrows = n + 56
    sc = _make_sc_kernel(n, ep, nrows, nchunks)
    num6, den6 = sc(xl6f, xr6f, srcg, dstg, attsc)

    risk_vector = _run_finalize(num6, den6, xl6, xr6, params, n)
    return (risk_vector, lat_a, lat_b, lat_c)


# speedup vs baseline: 3.2958x; 1.8422x over previous
import functools

import jax
import jax.numpy as jnp
from jax import lax
from jax.experimental import pallas as pl
from jax.experimental.pallas import tpu as pltpu
from jax.experimental.pallas import tpu_sc as plsc

RISKS = ("risk_a", "risk_b", "risk_c")
IN_MAP = {"risk_a": (0, 1), "risk_b": (1, 2), "risk_c": (0, 2, 3)}
D = 32
H = 2
NS = 16
LANES = 16

BN = 1000


def _gelu(x):
    return 0.5 * x * (1.0 + lax.erf(x * (2.0 ** -0.5)))


def _lrelu(x):
    return jnp.where(x >= 0, x, 0.2 * x)


def _prepass_body(m1, m2, m3, m4,
                  w1a, b1a, w2a, b2a, w1b, b1b, w2b, b2b, w1c, b1c, w2c, b2c,
                  wl, wr,
                  lat_a, lat_b, lat_c, xl6, xr6):
    mods = (m1[...], m2[...], m3[...], m4[...])
    encs = ((w1a, b1a, w2a, b2a), (w1b, b1b, w2b, b2b), (w1c, b1c, w2c, b2c))
    lat_refs = (lat_a, lat_b, lat_c)
    for r in range(3):
        w1, b1, w2, b2 = encs[r]
        h = b1[...]
        for k, col in enumerate(IN_MAP[RISKS[r]]):
            h = h + mods[col] * w1[k:k + 1, :]
        h = _gelu(h)
        lat = jnp.dot(h, w2[...]) + b2[...]
        lat_refs[r][...] = lat
        xl = jnp.dot(lat, wl[...])
        xr = jnp.dot(lat, wr[...])
        xl6[2 * r, :, :] = xl[:, :D]
        xl6[2 * r + 1, :, :] = xl[:, D:]
        xr6[2 * r, :, :] = xr[:, :D]
        xr6[2 * r + 1, :, :] = xr[:, D:]


def _run_prepass(mods, params, n):
    grid = n // BN
    full = lambda shape: pl.BlockSpec(shape, lambda i: tuple(0 for _ in shape))
    in_specs = [pl.BlockSpec((BN, 1), lambda i: (i, 0))] * 4
    wargs = []
    for risk in RISKS:
        p = params[risk]
        wargs += [p["enc_W1"], p["enc_b1"].reshape(1, 64),
                  p["enc_W2"], p["enc_b2"].reshape(1, D)]
    g = params["gat"]
    wargs += [g["Wl"], g["Wr"]]
    in_specs += [full(w.shape) for w in wargs]
    out_shape = [jax.ShapeDtypeStruct((n, D), jnp.float32) for _ in range(3)]
    out_shape += [jax.ShapeDtypeStruct((6, n, D), jnp.float32)] * 2
    out_specs = [pl.BlockSpec((BN, D), lambda i: (i, 0))] * 3
    out_specs += [pl.BlockSpec((6, BN, D), lambda i: (0, i, 0))] * 2
    return pl.pallas_call(
        _prepass_body,
        grid=(grid,),
        in_specs=in_specs,
        out_specs=out_specs,
        out_shape=out_shape,
    )(*mods, *wargs)


def _make_sc_kernel(n, ep, nrows, nchunks):
    rows_pt = nchunks
    stripe = 3120
    mesh = plsc.VectorSubcoreMesh(core_axis_name="c", subcore_axis_name="s")
    f32 = jnp.float32

    @functools.partial(
        pl.kernel,
        mesh=mesh,
        compiler_params=pltpu.CompilerParams(needs_layout_passes=False,
                                             use_tc_tiling_on_sc=False),
        out_type=[jax.ShapeDtypeStruct((6 * n, D), f32),
                  jax.ShapeDtypeStruct((6 * n,), f32)],
        scratch_types=[
            pltpu.VMEM((2, 128), jnp.int32),
            pltpu.VMEM((2, 128), jnp.int32),
            pltpu.VMEM((2, 128), jnp.int32),
            pltpu.VMEM((2, 128, D), f32),
            pltpu.VMEM((2, 128, D), f32),
            pltpu.VMEM((2, 128, D), f32),
            pltpu.VMEM((2, 128), f32),
            pltpu.VMEM((D,), f32),
            pltpu.VMEM((40, D), f32),
            pltpu.VMEM((48,), f32),
            pltpu.VMEM_SHARED((nrows, D), f32),
            pltpu.VMEM_SHARED((nrows,), f32),
            pltpu.SemaphoreType.DMA,
            pltpu.SemaphoreType.DMA,
            pltpu.SemaphoreType.DMA,
            pltpu.SemaphoreType.DMA,
            pltpu.SemaphoreType.DMA,
        ],
    )
    def sc_edges(xl6, xr6, srcg, dstg, attsc,
                 num_out, den_out,
                 idx_s, idx_dg, sidx, xlr, xrr, outr, exb, attv,
                 zbuf, zden, num_sh, den_sh, sem1, sem2, sem3, sem4, sem5):
        c = lax.axis_index("c")
        s = lax.axis_index("s")
        z16 = jnp.zeros((LANES,), f32)

        for i in range(40):
            zbuf[i, pl.ds(0, LANES)] = z16
            zbuf[i, pl.ds(LANES, LANES)] = z16
        for i in range(3):
            zden[pl.ds(i * LANES, LANES)] = z16

        pltpu.sync_copy(attsc.at[c], attv)

        lane = lax.iota(jnp.int32, LANES)
        zero_base = stripe * s

        def zero_body(k, _):
            pltpu.sync_copy(zbuf, num_sh.at[pl.ds(zero_base + k * 40, 40)])
            pltpu.sync_copy(zden.at[pl.ds(0, 40)],
                            den_sh.at[pl.ds(zero_base + k * 40, 40)])
            return 0

        att0 = attv[pl.ds(0, LANES)]
        att1 = attv[pl.ds(LANES, LANES)]
        last = jnp.full((LANES,), LANES - 1, jnp.int32)
        m0 = lane == 0

        for r in range(3):
            lax.fori_loop(0, 80, zero_body, 0)
            plsc.subcore_barrier()

            t_vec = jnp.full((LANES,), (2 * r) * n, jnp.int32) + c * n
            att0 = attv[pl.ds(0, LANES)]
            att1 = attv[pl.ds(LANES, LANES)]
            row_base = s * rows_pt

            pltpu.sync_copy(srcg.at[r, c, row_base], idx_s.at[0])
            pltpu.sync_copy(dstg.at[r, c, row_base], idx_dg.at[0])
            pltpu.sync_copy(srcg.at[r, c, row_base + 1], idx_s.at[1])
            pltpu.sync_copy(dstg.at[r, c, row_base + 1], idx_dg.at[1])
            pltpu.async_copy(xl6.at[idx_s.at[0]], xlr.at[0], sem1)
            pltpu.async_copy(xr6.at[idx_dg.at[0]], xrr.at[0], sem2)

            def chunk_body(j, _):
                p = jnp.bitwise_and(j, 1)
                q = 1 - p
                pltpu.make_async_copy(xl6.at[idx_s.at[p]], xlr.at[p],
                                      sem1).wait()
                pltpu.make_async_copy(xr6.at[idx_dg.at[p]], xrr.at[p],
                                      sem2).wait()

                @pl.when(j >= 2)
                def _():
                    pltpu.make_async_copy(outr.at[p],
                                          num_sh.at[sidx.at[p]], sem3).wait()
                    pltpu.make_async_copy(exb.at[p],
                                          den_sh.at[sidx.at[p]], sem4).wait()

                def unoff_body(g, _):
                    sl = pl.ds(g * LANES, LANES)
                    sidx[p, sl] = idx_dg[p, sl] - t_vec
                    return 0

                lax.fori_loop(0, 8, unoff_body, 0)

                @pl.when(j + 2 < nchunks)
                def _():
                    pltpu.async_copy(srcg.at[r, c, row_base + j + 2],
                                     idx_s.at[p], sem5)
                    pltpu.async_copy(dstg.at[r, c, row_base + j + 2],
                                     idx_dg.at[p], sem5)

                @pl.when(j + 1 < nchunks)
                def _():
                    @pl.when(j >= 1)
                    def _():
                        pltpu.make_async_copy(srcg.at[r, c, row_base + j + 1],
                                              idx_s.at[q], sem5).wait()
                        pltpu.make_async_copy(dstg.at[r, c, row_base + j + 1],
                                              idx_dg.at[q], sem5).wait()
                    pltpu.async_copy(xl6.at[idx_s.at[q]], xlr.at[q], sem1)
                    pltpu.async_copy(xr6.at[idx_dg.at[q]], xrr.at[q], sem2)

                pp = jnp.zeros((LANES,), jnp.int32) + p

                @plsc.parallel_loop(0, 128, unroll=8)
                def edge_body(e):
                    xl0 = xlr[p, e, pl.ds(0, LANES)]
                    xl1 = xlr[p, e, pl.ds(LANES, LANES)]
                    xr0 = xrr[p, e, pl.ds(0, LANES)]
                    xr1 = xrr[p, e, pl.ds(LANES, LANES)]
                    z0 = xl0 + xr0
                    z1 = xl1 + xr1
                    t = (jnp.maximum(z0, 0.2 * z0) * att0
                         + jnp.maximum(z1, 0.2 * z1) * att1)
                    tot = jnp.take_along_axis(plsc.cumsum(t), last, axis=0)
                    ex = jnp.exp(tot)
                    outr[p, e, pl.ds(0, LANES)] = xl0 * ex
                    outr[p, e, pl.ds(LANES, LANES)] = xl1 * ex
                    plsc.store_scatter(
                        exb, [pp, jnp.full((LANES,), e, jnp.int32)],
                        ex, mask=m0)

                pltpu.async_copy(outr.at[p], num_sh.at[sidx.at[p]], sem3,
                                 add=True)
                pltpu.async_copy(exb.at[p], den_sh.at[sidx.at[p]], sem4,
                                 add=True)
                return 0

            lax.fori_loop(0, nchunks, chunk_body, 0)

            for k in (nchunks - 2, nchunks - 1):
                pk = k % 2
                pltpu.make_async_copy(outr.at[pk],
                                      num_sh.at[sidx.at[pk]], sem3).wait()
                pltpu.make_async_copy(exb.at[pk],
                                      den_sh.at[sidx.at[pk]], sem4).wait()

            plsc.subcore_barrier()

            t_off = (2 * r + c) * n
            b0 = stripe * s
            pltpu.sync_copy(num_sh.at[pl.ds(b0, stripe)],
                            num_out.at[pl.ds(t_off + b0, stripe)])
            pltpu.sync_copy(den_sh.at[pl.ds(b0, stripe)],
                            den_out.at[pl.ds(t_off + b0, stripe)])

            @pl.when(s == NS - 1)
            def _():
                pltpu.sync_copy(num_sh.at[pl.ds(15 * stripe + stripe, 80)],
                                num_out.at[pl.ds(t_off + 16 * stripe, 80)])
                pltpu.sync_copy(den_sh.at[pl.ds(15 * stripe + stripe, 80)],
                                den_out.at[pl.ds(t_off + 16 * stripe, 80)])

    return sc_edges


def _finalize_body(num, den, xl6, xr6, att64, bias,
                   hw1a, hb1a, hw2a, hb2a, hw1b, hb1b, hw2b, hb2b,
                   hw1c, hb1c, hw2c, hb2c, out):
    heads = ((hw1a, hb1a, hw2a, hb2a), (hw1b, hb1b, hw2b, hb2b),
             (hw1c, hb1c, hw2c, hb2c))
    preds = []
    for r in range(3):
        gs = []
        for h in range(H):
            t = 2 * r + h
            xl = xl6[t, :, :]
            z = xl + xr6[t, :, :]
            logit = jnp.sum(_lrelu(z) * att64[:, h * D:(h + 1) * D],
                            axis=1, keepdims=True)
            ex = jnp.exp(logit)
            dent = den[t, :, :] + ex
            numt = num[t, :, :] + xl * ex
            gs.append(numt / (dent + 1e-16))
        gout = 0.5 * (gs[0] + gs[1]) + bias[...]
        gact = _gelu(gout)
        hw1, hb1, hw2, hb2 = heads[r]
        h1 = _gelu(jnp.dot(gact, hw1[...]) + hb1[...])
        z2 = jnp.dot(h1, hw2[...]) + hb2[...]
        preds.append(jax.nn.sigmoid(z2))
    out[...] = jnp.concatenate(preds, axis=1)


def _run_finalize(num6, den6, xl6, xr6, params, n):
    grid = n // BN
    full = lambda shape: pl.BlockSpec(shape, lambda i: tuple(0 for _ in shape))
    g = params["gat"]
    att64 = g["att"].reshape(1, H * D)
    bias = g["bias"].reshape(1, D)
    wargs = [att64, bias]
    for risk in RISKS:
        p = params[risk]
        wargs += [p["head_W1"], p["head_b1"].reshape(1, D),
                  p["head_W2"], p["head_b2"].reshape(1, 1)]
    num = num6.reshape(6, n, D)
    den = den6.reshape(6, n, 1)
    in_specs = [pl.BlockSpec((6, BN, D), lambda i: (0, i, 0)),
                pl.BlockSpec((6, BN, 1), lambda i: (0, i, 0)),
                pl.BlockSpec((6, BN, D), lambda i: (0, i, 0)),
                pl.BlockSpec((6, BN, D), lambda i: (0, i, 0))]
    in_specs += [full(w.shape) for w in wargs]
    return pl.pallas_call(
        _finalize_body,
        grid=(grid,),
        in_specs=in_specs,
        out_specs=pl.BlockSpec((BN, 3), lambda i: (i, 0)),
        out_shape=jax.ShapeDtypeStruct((n, 3), jnp.float32),
    )(num, den, xl6.reshape(6, n, D), xr6.reshape(6, n, D), *wargs)


def kernel(mod1, mod2, mod3, mod4, edge_index, params):
    n = mod1.shape[0]
    e = edge_index.shape[1]

    lat_a, lat_b, lat_c, xl6, xr6 = _run_prepass(
        (mod1, mod2, mod3, mod4), params, n)

    nchunks = -(-e // (NS * 128))
    ep = NS * 128 * nchunks
    pad = ep - e
    src = jnp.concatenate([edge_index[0],
                           jnp.zeros((pad,), edge_index.dtype)])
    dst_raw = edge_index[1]
    dst = jnp.concatenate([dst_raw,
                           jnp.full((pad,), n, edge_index.dtype)])
    offs = (jnp.arange(6, dtype=jnp.int32) * n).reshape(3, 2, 1)
    srcg = (src[None, None, :] + offs).reshape(3, 2, ep // 128, 128)
    dstg = (dst[None, None, :] + offs).reshape(3, 2, ep // 128, 128)

    xl6f = jnp.concatenate([xl6.reshape(6 * n, D),
                            jnp.zeros((8, D), jnp.float32)])
    xr6f = jnp.concatenate([xr6.reshape(6 * n, D),
                            jnp.zeros((8, D), jnp.float32)])
    attsc = params["gat"]["att"]

    nrows = n + 56
    sc = _make_sc_kernel(n, ep, nrows, nchunks)
    num6, den6 = sc(xl6f, xr6f, srcg, dstg, attsc)

    risk_vector = _run_finalize(num6, den6, xl6, xr6, params, n)
    return (risk_vector, lat_a, lat_b, lat_c)
